# baseline (speedup 1.0000x reference)
import jax
import jax.numpy as jnp
from jax import lax
from jax.experimental import pallas as pl
from jax.experimental.pallas import tpu as pltpu
from jax.experimental.pallas import tpu_sc as plsc

NC, NS = 2, 16
NW = NC * NS
CH = 128
NB = 2
NB_S = 8


def _sc_mesh():
    return plsc.VectorSubcoreMesh(core_axis_name="c", subcore_axis_name="s")


def _rpt(n_nodes):
    return ((n_nodes + NS - 1) // NS + 127) // 128 * 128


def _deg_call(idxb, n_nodes):
    nblk = idxb.shape[0]
    ch_w = nblk // NW
    rpt = _rpt(n_nodes)
    n_pad = NS * rpt
    zeros = jnp.zeros((rpt,), jnp.float32)
    ones = jnp.ones((CH,), jnp.float32)

    def body(dst_hbm, zeros_hbm, ones_hbm, out_hbm, acc, stage, ones_v, *sl):
        idxs, sis, sss = sl[0:NB_S], sl[NB_S:2 * NB_S], sl[2 * NB_S:3 * NB_S]
        cid = lax.axis_index("c")
        sid = lax.axis_index("s")
        my = pl.ds(sid * rpt, rpt)
        pltpu.sync_copy(zeros_hbm, stage)
        pltpu.sync_copy(stage, acc.at[my])
        pltpu.sync_copy(ones_hbm, ones_v)
        plsc.subcore_barrier()
        base = (cid * NS + sid) * ch_w

        def quad(i, c):
            c0 = base + i * NB_S
            h_i = [pltpu.async_copy(dst_hbm.at[c0 + b], idxs[b], sis[b])
                   for b in range(NB_S)]
            h_s = []
            for b in range(NB_S):
                h_i[b].wait()
                h_s.append(pltpu.async_copy(
                    ones_v, acc.at[idxs[b].at[1]], sss[b], add=True))
            for h in h_s:
                h.wait()
            return c

        lax.fori_loop(0, ch_w // NB_S, quad, 0)
        plsc.subcore_barrier()
        pltpu.sync_copy(acc.at[my], stage)
        pltpu.sync_copy(stage, out_hbm.at[pl.ds((cid * NS + sid) * rpt, rpt)])

    f = pl.kernel(
        body,
        out_type=jax.ShapeDtypeStruct((NC * n_pad,), jnp.float32),
        mesh=_sc_mesh(),
        scratch_types=[
            pltpu.VMEM_SHARED((n_pad,), jnp.float32),
            pltpu.VMEM((rpt,), jnp.float32),
            pltpu.VMEM((CH,), jnp.float32),
        ] + [pltpu.VMEM((2, CH), jnp.int32) for _ in range(NB_S)]
          + [pltpu.SemaphoreType.DMA for _ in range(2 * NB_S)],
        compiler_params=pltpu.CompilerParams(use_tc_tiling_on_sc=False),
    )
    return f(idxb, zeros, ones), n_pad


def _scatter_add_call(g, idxb, n_nodes, nb):
    d = g.shape[1]
    nblk = idxb.shape[0]
    ch_w = nblk // NW
    nstage = 8
    rpt = _rpt(n_nodes)
    spt = rpt // nstage
    n_pad = NS * rpt
    zeros = jnp.zeros((spt, d), jnp.float32)

    def body(g_hbm, idx_hbm, zeros_hbm, out_hbm, acc, stage, *sl):
        idxs, bufs = sl[0:nb], sl[nb:2 * nb]
        sis = sl[2 * nb:3 * nb]
        sgs = sl[3 * nb:4 * nb]
        sss = sl[4 * nb:5 * nb]
        cid = lax.axis_index("c")
        sid = lax.axis_index("s")
        pltpu.sync_copy(zeros_hbm, stage)
        for k in range(nstage):
            pltpu.sync_copy(stage, acc.at[pl.ds(sid * rpt + k * spt, spt)])
        plsc.subcore_barrier()
        base = (cid * NS + sid) * ch_w

        def quad(i, c):
            c0 = base + i * nb
            h_i = [pltpu.async_copy(idx_hbm.at[c0 + b], idxs[b], sis[b])
                   for b in range(nb)]
            h_g = []
            for b in range(nb):
                h_i[b].wait()
                h_g.append(pltpu.async_copy(
                    g_hbm.at[idxs[b].at[0]], bufs[b], sgs[b]))
            h_s = []
            for b in range(nb):
                h_g[b].wait()
                h_s.append(pltpu.async_copy(
                    bufs[b], acc.at[idxs[b].at[1]], sss[b], add=True))
            for h in h_s:
                h.wait()
            return c

        lax.fori_loop(0, ch_w // nb, quad, 0)
        plsc.subcore_barrier()
        for k in range(nstage):
            pltpu.sync_copy(acc.at[pl.ds(sid * rpt + k * spt, spt)], stage)
            pltpu.sync_copy(
                stage,
                out_hbm.at[pl.ds((cid * NS + sid) * rpt + k * spt, spt)])

    f = pl.kernel(
        body,
        out_type=jax.ShapeDtypeStruct((NC * n_pad, d), jnp.float32),
        mesh=_sc_mesh(),
        scratch_types=[
            pltpu.VMEM_SHARED((n_pad, d), jnp.float32),
            pltpu.VMEM((spt, d), jnp.float32),
        ] + [pltpu.VMEM((2, CH), jnp.int32) for _ in range(nb)]
          + [pltpu.VMEM((CH, d), jnp.float32) for _ in range(nb)]
          + [pltpu.SemaphoreType.DMA for _ in range(3 * nb)],
        compiler_params=pltpu.CompilerParams(use_tc_tiling_on_sc=False),
    )
    return f(g, idxb, zeros), n_pad


def _mm(x_ref, w1_ref, h_ref):
    h_ref[...] = jnp.dot(x_ref[...], w1_ref[...],
                         preferred_element_type=jnp.float32)


def _scale(d2_ref, h_ref, g_ref, dis_ref):
    dsum = d2_ref[:, 0:1] + d2_ref[:, 1:2]
    pos = dsum > 0
    dis = jnp.where(pos, lax.rsqrt(jnp.where(pos, dsum, 1.0)), 0.0)
    g_ref[...] = h_ref[...] * dis
    dis_ref[...] = dis


def _mid(a0_ref, a1_ref, dis_ref, b1_ref, w2_ref, g2_ref):
    dis = dis_ref[...]
    z = jnp.maximum((a0_ref[...] + a1_ref[...]) * dis + b1_ref[...], 0.0)
    h2 = jnp.dot(z, w2_ref[...], preferred_element_type=jnp.float32)
    g2_ref[...] = h2 * dis


def _fin(a0_ref, a1_ref, dis_ref, b2_ref, out_ref):
    y = (a0_ref[...] + a1_ref[...]) * dis_ref[...] + b2_ref[...]
    m = jnp.max(y, axis=1, keepdims=True)
    e = jnp.exp(y - m)
    s = jnp.sum(e, axis=1, keepdims=True)
    out_ref[...] = y - m - jnp.log(s)


def kernel(x, edge_index, W1, b1, W2, b2):
    n, f_in = x.shape
    hid = W1.shape[1]
    out_dim = W2.shape[1]
    dpad = 8
    E = edge_index.shape[1]

    blk = NW * CH * NB_S
    e_pad = (E + blk - 1) // blk * blk
    pad_rows = NS * _rpt(n) - n
    pad_iota = jnp.arange(e_pad - E, dtype=jnp.int32)
    srcp = jnp.concatenate([edge_index[0], pad_iota % n])
    dstp = jnp.concatenate([edge_index[1], n + pad_iota % pad_rows])
    idxb = jnp.stack(
        [srcp.reshape(-1, CH), dstp.reshape(-1, CH)], axis=1)

    degs, n_pad1 = _deg_call(idxb, n)
    h = pl.pallas_call(
        _mm, out_shape=jax.ShapeDtypeStruct((n, hid), jnp.float32))(x, W1)

    d2 = degs.reshape(NC, n_pad1)[:, :n].T
    g1, dis = pl.pallas_call(
        _scale,
        out_shape=(jax.ShapeDtypeStruct((n, hid), jnp.float32),
                   jax.ShapeDtypeStruct((n, 1), jnp.float32)),
    )(d2, h)

    acc1, n_pad = _scatter_add_call(g1, idxb, n, NB)
    a = acc1.reshape(NC, n_pad, hid)

    w2p = jnp.concatenate(
        [W2, jnp.zeros((hid, dpad - out_dim), jnp.float32)], axis=1)
    g2 = pl.pallas_call(
        _mid,
        out_shape=jax.ShapeDtypeStruct((n, dpad), jnp.float32),
    )(a[0, :n], a[1, :n], dis, b1.reshape(1, hid), w2p)

    acc2, n_pad2 = _scatter_add_call(g2, idxb, n, NB_S)
    a2 = acc2.reshape(NC, n_pad2, dpad)

    b2p = jnp.concatenate(
        [b2, jnp.full((dpad - out_dim,), -1e30, jnp.float32)]).reshape(1, dpad)
    out8 = pl.pallas_call(
        _fin,
        out_shape=jax.ShapeDtypeStruct((n, dpad), jnp.float32),
    )(a2[0, :n], a2[1, :n], dis, b2p)

    out = out8[:, :out_dim]
    return (out, out, out, out)


# speedup vs baseline: 1.0426x; 1.0426x over previous
import jax
import jax.numpy as jnp
from jax import lax
from jax.experimental import pallas as pl
from jax.experimental.pallas import tpu as pltpu
from jax.experimental.pallas import tpu_sc as plsc

NC, NS = 2, 16
NW = NC * NS
CH = 128
NB = 2
NB_S = 8


def _sc_mesh():
    return plsc.VectorSubcoreMesh(core_axis_name="c", subcore_axis_name="s")


def _rpt(n_nodes):
    return ((n_nodes + NS - 1) // NS + 127) // 128 * 128


def _deg_call(idxb, n_nodes):
    nblk = idxb.shape[0]
    ch_w = nblk // NW
    rpt = _rpt(n_nodes)
    n_pad = NS * rpt
    zeros = jnp.zeros((rpt,), jnp.float32)
    ones = jnp.ones((CH,), jnp.float32)

    def body(dst_hbm, zeros_hbm, ones_hbm, out_hbm, acc, stage, ones_v, *sl):
        idxs, sis, sss = sl[0:NB_S], sl[NB_S:2 * NB_S], sl[2 * NB_S:3 * NB_S]
        cid = lax.axis_index("c")
        sid = lax.axis_index("s")
        my = pl.ds(sid * rpt, rpt)
        pltpu.sync_copy(zeros_hbm, stage)
        pltpu.sync_copy(stage, acc.at[my])
        pltpu.sync_copy(ones_hbm, ones_v)
        plsc.subcore_barrier()
        base = (cid * NS + sid) * ch_w

        def quad(i, c):
            c0 = base + i * NB_S
            h_i = [pltpu.async_copy(dst_hbm.at[c0 + b], idxs[b], sis[b])
                   for b in range(NB_S)]
            h_s = []
            for b in range(NB_S):
                h_i[b].wait()
                h_s.append(pltpu.async_copy(
                    ones_v, acc.at[idxs[b].at[1]], sss[b], add=True))
            for h in h_s:
                h.wait()
            return c

        lax.fori_loop(0, ch_w // NB_S, quad, 0)
        plsc.subcore_barrier()
        pltpu.sync_copy(acc.at[my], stage)
        pltpu.sync_copy(stage, out_hbm.at[pl.ds((cid * NS + sid) * rpt, rpt)])

    f = pl.kernel(
        body,
        out_type=jax.ShapeDtypeStruct((NC * n_pad,), jnp.float32),
        mesh=_sc_mesh(),
        scratch_types=[
            pltpu.VMEM_SHARED((n_pad,), jnp.float32),
            pltpu.VMEM((rpt,), jnp.float32),
            pltpu.VMEM((CH,), jnp.float32),
        ] + [pltpu.VMEM((2, CH), jnp.int32) for _ in range(NB_S)]
          + [pltpu.SemaphoreType.DMA for _ in range(2 * NB_S)],
        compiler_params=pltpu.CompilerParams(use_tc_tiling_on_sc=False),
    )
    return f(idxb, zeros, ones), n_pad


def _scatter_add_call(g, idxb, n_nodes, nb):
    d = g.shape[1]
    nblk = idxb.shape[0]
    ch_w = nblk // NW
    nstage = 8
    rpt = _rpt(n_nodes)
    spt = rpt // nstage
    n_pad = NS * rpt
    zeros = jnp.zeros((spt, d), jnp.float32)

    def body(g_hbm, idx_hbm, zeros_hbm, out_hbm, acc, stage, *sl):
        idxs, bufs = sl[0:nb], sl[nb:2 * nb]
        sis = sl[2 * nb:3 * nb]
        sgs = sl[3 * nb:4 * nb]
        sss = sl[4 * nb:5 * nb]
        cid = lax.axis_index("c")
        sid = lax.axis_index("s")
        pltpu.sync_copy(zeros_hbm, stage)
        for k in range(nstage):
            pltpu.sync_copy(stage, acc.at[pl.ds(sid * rpt + k * spt, spt)])
        plsc.subcore_barrier()
        base = (cid * NS + sid) * ch_w

        def quad(i, c):
            c0 = base + i * nb
            h_i = [pltpu.async_copy(idx_hbm.at[c0 + b], idxs[b], sis[b])
                   for b in range(nb)]
            h_g = []
            for b in range(nb):
                h_i[b].wait()
                h_g.append(pltpu.async_copy(
                    g_hbm.at[idxs[b].at[0]], bufs[b], sgs[b]))
            h_s = []
            for b in range(nb):
                h_g[b].wait()
                h_s.append(pltpu.async_copy(
                    bufs[b], acc.at[idxs[b].at[1]], sss[b], add=True))
            for h in h_s:
                h.wait()
            return c

        lax.fori_loop(0, ch_w // nb, quad, 0)
        plsc.subcore_barrier()
        for k in range(nstage):
            pltpu.sync_copy(acc.at[pl.ds(sid * rpt + k * spt, spt)], stage)
            pltpu.sync_copy(
                stage,
                out_hbm.at[pl.ds((cid * NS + sid) * rpt + k * spt, spt)])

    f = pl.kernel(
        body,
        out_type=jax.ShapeDtypeStruct((NC * n_pad, d), jnp.float32),
        mesh=_sc_mesh(),
        scratch_types=[
            pltpu.VMEM_SHARED((n_pad, d), jnp.float32),
            pltpu.VMEM((spt, d), jnp.float32),
        ] + [pltpu.VMEM((2, CH), jnp.int32) for _ in range(nb)]
          + [pltpu.VMEM((CH, d), jnp.float32) for _ in range(nb)]
          + [pltpu.SemaphoreType.DMA for _ in range(3 * nb)],
        compiler_params=pltpu.CompilerParams(use_tc_tiling_on_sc=False),
    )
    return f(g, idxb, zeros), n_pad


def _mm(x_ref, w1_ref, h_ref):
    h_ref[...] = jnp.dot(x_ref[...], w1_ref[...],
                         preferred_element_type=jnp.float32)


def _scale(d2_ref, h_ref, g_ref, dis_ref):
    dsum = d2_ref[:, 0:1] + d2_ref[:, 1:2]
    pos = dsum > 0
    dis = jnp.where(pos, lax.rsqrt(jnp.where(pos, dsum, 1.0)), 0.0)
    g_ref[...] = h_ref[...] * dis
    dis_ref[...] = dis


def _mid(n, n_pad):
    def f(acc_ref, dis_ref, b1_ref, w2_ref, g2_ref):
        dis = dis_ref[...]
        s = acc_ref[pl.ds(0, n), :] + acc_ref[pl.ds(n_pad, n), :]
        z = jnp.maximum(s * dis + b1_ref[...], 0.0)
        h2 = jnp.dot(z, w2_ref[...], preferred_element_type=jnp.float32)
        g2_ref[...] = h2 * dis
    return f


def _fin(n, n_pad, out_dim):
    def f(acc_ref, dis_ref, b2_ref, out_ref):
        s = acc_ref[pl.ds(0, n), :] + acc_ref[pl.ds(n_pad, n), :]
        y = s * dis_ref[...] + b2_ref[...]
        m = jnp.max(y, axis=1, keepdims=True)
        e = jnp.exp(y - m)
        t = jnp.sum(e, axis=1, keepdims=True)
        out_ref[...] = (y - m - jnp.log(t))[:, :out_dim]
    return f


def kernel(x, edge_index, W1, b1, W2, b2):
    n, f_in = x.shape
    hid = W1.shape[1]
    out_dim = W2.shape[1]
    dpad = 8
    E = edge_index.shape[1]

    blk = NW * CH * NB_S
    e_pad = (E + blk - 1) // blk * blk
    pad_rows = NS * _rpt(n) - n
    pad_iota = jnp.arange(e_pad - E, dtype=jnp.int32)
    srcp = jnp.concatenate([edge_index[0], pad_iota % n])
    dstp = jnp.concatenate([edge_index[1], n + pad_iota % pad_rows])
    idxb = jnp.stack(
        [srcp.reshape(-1, CH), dstp.reshape(-1, CH)], axis=1)

    degs, n_pad1 = _deg_call(idxb, n)
    h = pl.pallas_call(
        _mm, out_shape=jax.ShapeDtypeStruct((n, hid), jnp.float32))(x, W1)

    d2 = degs.reshape(NC, n_pad1)[:, :n].T
    g1, dis = pl.pallas_call(
        _scale,
        out_shape=(jax.ShapeDtypeStruct((n, hid), jnp.float32),
                   jax.ShapeDtypeStruct((n, 1), jnp.float32)),
    )(d2, h)

    acc1, n_pad = _scatter_add_call(g1, idxb, n, NB)

    w2p = jnp.concatenate(
        [W2, jnp.zeros((hid, dpad - out_dim), jnp.float32)], axis=1)
    g2 = pl.pallas_call(
        _mid(n, n_pad),
        out_shape=jax.ShapeDtypeStruct((n, dpad), jnp.float32),
    )(acc1, dis, b1.reshape(1, hid), w2p)

    acc2, n_pad2 = _scatter_add_call(g2, idxb, n, NB_S)

    b2p = jnp.concatenate(
        [b2, jnp.full((dpad - out_dim,), -1e30, jnp.float32)]).reshape(1, dpad)
    out = pl.pallas_call(
        _fin(n, n_pad2, out_dim),
        out_shape=jax.ShapeDtypeStruct((n, out_dim), jnp.float32),
    )(acc2, dis, b2p)

    return (out, out, out, out)
